# baseline (speedup 1.0000x reference)
import functools

import jax
import jax.numpy as jnp
from jax import lax
from jax.experimental import pallas as pl
from jax.experimental.pallas import tpu as pltpu
from jax.experimental.pallas import tpu_sc as plsc

L = 16
NC = 2
NS = 16
NW = NC * NS
B = 16384
D = 32
BPW = B // NW
CHUNK = 128
NCHUNK = BPW // CHUNK

_mesh = plsc.VectorSubcoreMesh(core_axis_name="c", subcore_axis_name="s")


@functools.partial(
    pl.kernel,
    mesh=_mesh,
    out_type=jax.ShapeDtypeStruct((B,), jnp.float32),
    compiler_params=pltpu.CompilerParams(
        needs_layout_passes=False, use_tc_tiling_on_sc=False),
    scratch_types=[
        pltpu.VMEM((NCHUNK, CHUNK), jnp.int32),
        pltpu.VMEM((NCHUNK, CHUNK), jnp.int32),
        pltpu.VMEM((BPW, D), jnp.float32),
        pltpu.VMEM((BPW, D), jnp.float32),
        pltpu.VMEM((BPW,), jnp.float32),
        pltpu.SemaphoreType.DMA,
    ],
)
def _mf_sc(user_hbm, item_hbm, utab_hbm, itab_hbm, out_hbm,
           uidx_v, iidx_v, urows_v, irows_v, out_v, sem):
    wid = lax.axis_index("s") * NC + lax.axis_index("c")
    base = wid * BPW

    pltpu.sync_copy(user_hbm.at[wid], uidx_v)
    pltpu.sync_copy(item_hbm.at[wid], iidx_v)

    copies = []
    for j in range(NCHUNK):
        copies.append(pltpu.async_copy(
            utab_hbm.at[uidx_v.at[j]], urows_v.at[pl.ds(j * CHUNK, CHUNK)], sem))
        copies.append(pltpu.async_copy(
            itab_hbm.at[iidx_v.at[j]], irows_v.at[pl.ds(j * CHUNK, CHUNK)], sem))
    for c in copies:
        c.wait()

    lanes = lax.iota(jnp.int32, L)

    def blk_body(blk, carry):
        row = blk * L + lanes
        acc = jnp.zeros((L,), jnp.float32)
        for dcol in range(D):
            col = jnp.full((L,), dcol, jnp.int32)
            u = plsc.load_gather(urows_v, [row, col])
            v = plsc.load_gather(irows_v, [row, col])
            acc = acc + u * v
        out_v[pl.ds(blk * L, L)] = acc
        return carry

    lax.fori_loop(0, BPW // L, blk_body, 0)

    pltpu.sync_copy(out_v, out_hbm.at[pl.ds(base, BPW)])


def kernel(user, item, user_table, item_table):
    u3 = user.reshape(NW, NCHUNK, CHUNK)
    i3 = item.reshape(NW, NCHUNK, CHUNK)
    return _mf_sc(u3, i3, user_table, item_table)


# speedup vs baseline: 1.0000x; 1.0000x over previous
import functools

import jax
import jax.numpy as jnp
from jax import lax
from jax.experimental import pallas as pl
from jax.experimental.pallas import tpu as pltpu
from jax.experimental.pallas import tpu_sc as plsc

L = 16
NC = 2
NS = 16
NW = NC * NS
B = 16384
D = 32
RPV = 128 // D
BPW = B // NW
CHUNK = 128
NCHUNK = BPW // CHUNK
BLK = CHUNK // L

_mesh = plsc.VectorSubcoreMesh(core_axis_name="c", subcore_axis_name="s")


@functools.partial(
    pl.kernel,
    mesh=_mesh,
    out_type=jax.ShapeDtypeStruct((B,), jnp.float32),
    compiler_params=pltpu.CompilerParams(needs_layout_passes=False),
    scratch_types=[
        pltpu.VMEM((NCHUNK, CHUNK), jnp.int32),
        pltpu.VMEM((NCHUNK, CHUNK), jnp.int32),
        pltpu.VMEM((NCHUNK, CHUNK), jnp.int32),
        pltpu.VMEM((NCHUNK, CHUNK), jnp.int32),
        pltpu.VMEM((2, CHUNK, 128), jnp.float32),
        pltpu.VMEM((2, CHUNK, 128), jnp.float32),
        pltpu.VMEM((BPW,), jnp.float32),
        pltpu.SemaphoreType.DMA,
        pltpu.SemaphoreType.DMA,
    ],
)
def _mf_sc(user_hbm, item_hbm, utab_hbm, itab_hbm, out_hbm,
           uidx_v, iidx_v, urow_v, irow_v, ubuf, ibuf, out_v, sem0, sem1):
    wid = lax.axis_index("s") * NC + lax.axis_index("c")
    base = wid * BPW

    pltpu.sync_copy(user_hbm.at[wid], uidx_v)
    pltpu.sync_copy(item_hbm.at[wid], iidx_v)

    for c in range(NCHUNK):
        for t in range(BLK):
            s = pl.ds(t * L, L)
            urow_v[c, s] = lax.shift_right_logical(uidx_v[c, s], 2)
            irow_v[c, s] = lax.shift_right_logical(iidx_v[c, s], 2)

    sems = (sem0, sem1)

    def start(c):
        slot = c % 2
        return (
            pltpu.async_copy(utab_hbm.at[urow_v.at[c]], ubuf.at[slot], sems[slot]),
            pltpu.async_copy(itab_hbm.at[irow_v.at[c]], ibuf.at[slot], sems[slot]),
        )

    lanes = lax.iota(jnp.int32, L)
    pending = start(0)
    for c in range(NCHUNK):
        if c + 1 < NCHUNK:
            nxt = start(c + 1)
        for cp in pending:
            cp.wait()
        slot = c % 2

        ub, ib = ubuf.at[slot], ibuf.at[slot]

        def blk_body(b, carry, c=c, ub=ub, ib=ib):
            s = pl.ds(b * L, L)
            row = b * L + lanes
            cu = (uidx_v[c, s] & 3) << 5
            ci = (iidx_v[c, s] & 3) << 5
            acc = jnp.zeros((L,), jnp.float32)
            for _ in range(D):
                u = plsc.load_gather(ub, [row, cu])
                v = plsc.load_gather(ib, [row, ci])
                acc = acc + u * v
                cu = cu + 1
                ci = ci + 1
            out_v[pl.ds(c * CHUNK + b * L, L)] = acc
            return carry

        lax.fori_loop(0, BLK, blk_body, 0)
        if c + 1 < NCHUNK:
            pending = nxt

    pltpu.sync_copy(out_v, out_hbm.at[pl.ds(base, BPW)])


def kernel(user, item, user_table, item_table):
    u3 = user.reshape(NW, NCHUNK, CHUNK)
    i3 = item.reshape(NW, NCHUNK, CHUNK)
    ut = user_table.reshape(user_table.shape[0] // RPV, RPV * D)
    it = item_table.reshape(item_table.shape[0] // RPV, RPV * D)
    return _mf_sc(u3, i3, ut, it)


# speedup vs baseline: 1.0015x; 1.0015x over previous
import functools

import jax
import jax.numpy as jnp
from jax import lax
from jax.experimental import pallas as pl
from jax.experimental.pallas import tpu as pltpu
from jax.experimental.pallas import tpu_sc as plsc

L = 16
NC = 2
NS = 16
NW = NC * NS
B = 16384
D = 32
RPV = 128 // D
BPW = B // NW
CHUNK = 128
NCHUNK = BPW // CHUNK
BLK = CHUNK // L

_mesh = plsc.VectorSubcoreMesh(core_axis_name="c", subcore_axis_name="s")


@functools.partial(
    pl.kernel,
    mesh=_mesh,
    out_type=jax.ShapeDtypeStruct((B,), jnp.float32),
    compiler_params=pltpu.CompilerParams(
        needs_layout_passes=False, use_tc_tiling_on_sc=True),
    scratch_types=[
        pltpu.VMEM((NCHUNK, CHUNK), jnp.int32),
        pltpu.VMEM((NCHUNK, CHUNK), jnp.int32),
        pltpu.VMEM((NCHUNK, CHUNK), jnp.int32),
        pltpu.VMEM((NCHUNK, CHUNK), jnp.int32),
        pltpu.VMEM((2, CHUNK, 128), jnp.float32),
        pltpu.VMEM((2, CHUNK, 128), jnp.float32),
        pltpu.VMEM((BPW,), jnp.float32),
        pltpu.SemaphoreType.DMA,
        pltpu.SemaphoreType.DMA,
    ],
)
def _mf_sc(user_hbm, item_hbm, utab_hbm, itab_hbm, out_hbm,
           uidx_v, iidx_v, urow_v, irow_v, ubuf, ibuf, out_v, sem0, sem1):
    wid = lax.axis_index("s") * NC + lax.axis_index("c")
    base = wid * BPW

    pltpu.sync_copy(user_hbm.at[wid], uidx_v)
    pltpu.sync_copy(item_hbm.at[wid], iidx_v)

    for c in range(NCHUNK):
        for t in range(BLK):
            s = pl.ds(t * L, L)
            urow_v[c, s] = lax.shift_right_logical(uidx_v[c, s], 2)
            irow_v[c, s] = lax.shift_right_logical(iidx_v[c, s], 2)

    sems = (sem0, sem1)

    def start(c):
        slot = c % 2
        return (
            pltpu.async_copy(utab_hbm.at[urow_v.at[c]], ubuf.at[slot], sems[slot]),
            pltpu.async_copy(itab_hbm.at[irow_v.at[c]], ibuf.at[slot], sems[slot]),
        )

    lanes = lax.iota(jnp.int32, L)
    pending = start(0)
    for c in range(NCHUNK):
        if c + 1 < NCHUNK:
            nxt = start(c + 1)
        for cp in pending:
            cp.wait()
        slot = c % 2

        ub, ib = ubuf.at[slot], ibuf.at[slot]

        def blk_body(b, carry, c=c, ub=ub, ib=ib):
            s = pl.ds(b * L, L)
            row = b * L + lanes
            cu = (uidx_v[c, s] & 3) << 5
            ci = (iidx_v[c, s] & 3) << 5
            acc = jnp.zeros((L,), jnp.float32)
            for _ in range(D):
                u = plsc.load_gather(ub, [row, cu])
                v = plsc.load_gather(ib, [row, ci])
                acc = acc + u * v
                cu = cu + 1
                ci = ci + 1
            out_v[pl.ds(c * CHUNK + b * L, L)] = acc
            return carry

        lax.fori_loop(0, BLK, blk_body, 0)
        if c + 1 < NCHUNK:
            pending = nxt

    pltpu.sync_copy(out_v, out_hbm.at[pl.ds(base, BPW)])


def kernel(user, item, user_table, item_table):
    u3 = user.reshape(NW, NCHUNK, CHUNK)
    i3 = item.reshape(NW, NCHUNK, CHUNK)
    ut = user_table.reshape(user_table.shape[0] // RPV, RPV * D)
    it = item_table.reshape(item_table.shape[0] // RPV, RPV * D)
    return _mf_sc(u3, i3, ut, it)


# speedup vs baseline: 3.6994x; 3.6940x over previous
import functools

import jax
import jax.numpy as jnp
from jax import lax
from jax.experimental import pallas as pl
from jax.experimental.pallas import tpu as pltpu
from jax.experimental.pallas import tpu_sc as plsc

L = 16
NC = 2
NS = 16
NW = NC * NS
B = 16384
D = 32
BPW = B // NW
G = 4
NG = BPW // G
WIN = 128

_mesh = plsc.VectorSubcoreMesh(core_axis_name="c", subcore_axis_name="s")


@functools.partial(
    pl.kernel,
    mesh=_mesh,
    out_type=jax.ShapeDtypeStruct((B,), jnp.float32),
    compiler_params=pltpu.CompilerParams(
        needs_layout_passes=False, use_tc_tiling_on_sc=True),
    scratch_types=[
        pltpu.VMEM((BPW,), jnp.int32),
        pltpu.VMEM((BPW,), jnp.int32),
        pltpu.VMEM((2, G, D, WIN), jnp.float32),
        pltpu.VMEM((2, G, D, WIN), jnp.float32),
        pltpu.VMEM((BPW,), jnp.float32),
        pltpu.SemaphoreType.DMA,
        pltpu.SemaphoreType.DMA,
    ],
)
def _mf_sc(user_hbm, item_hbm, utab_hbm, itab_hbm, out_hbm,
           uidx_v, iidx_v, ubuf, ibuf, out_v, sem0, sem1):
    wid = lax.axis_index("s") * NC + lax.axis_index("c")
    base = wid * BPW

    pltpu.sync_copy(user_hbm.at[wid], uidx_v)
    pltpu.sync_copy(item_hbm.at[wid], iidx_v)

    lanes = lax.iota(jnp.int32, L)
    c_lo = lanes
    c_hi = lanes + L
    sems = (sem0, sem1)

    def fire(t, slab):
        sem = sems[slab]
        gbase = t * G
        ruv = uidx_v[pl.ds(gbase, L)]
        riv = iidx_v[pl.ds(gbase, L)]
        cu = ruv & ~(WIN - 1)
        ci = riv & ~(WIN - 1)
        for k in range(G):
            cuk = pl.multiple_of(cu[k], WIN)
            cik = pl.multiple_of(ci[k], WIN)
            pltpu.async_copy(
                utab_hbm.at[:, pl.ds(cuk, WIN)], ubuf.at[slab, k], sem)
            pltpu.async_copy(
                itab_hbm.at[:, pl.ds(cik, WIN)], ibuf.at[slab, k], sem)

    def wait(slab):
        for _ in range(2 * G):
            pltpu.make_async_copy(
                utab_hbm.at[:, pl.ds(0, WIN)], ubuf.at[0, 0], sems[slab]
            ).wait()

    def compute(t, slab, acc):
        gbase = t * G
        ruv = uidx_v[pl.ds(gbase, L)]
        riv = iidx_v[pl.ds(gbase, L)]
        rsu = ruv & (WIN - 1)
        rsi = riv & (WIN - 1)
        lane0 = (lax.rem(t, 4) * G).astype(jnp.int32)
        for k in range(G):
            su = jnp.full((L,), rsu[k], jnp.int32)
            si = jnp.full((L,), rsi[k], jnp.int32)
            u0 = plsc.load_gather(ubuf.at[slab, k], [c_lo, su])
            u1 = plsc.load_gather(ubuf.at[slab, k], [c_hi, su])
            i0 = plsc.load_gather(ibuf.at[slab, k], [c_lo, si])
            i1 = plsc.load_gather(ibuf.at[slab, k], [c_hi, si])
            p = u0 * i0 + u1 * i1
            dot = lax.reduce_sum(p, (0,))
            acc = jnp.where(lanes == lane0 + k, jnp.full((L,), dot), acc)
        out_v[pl.ds(lax.div(t, 4) * L, L)] = acc
        return jnp.where(jnp.full((L,), lax.rem(t, 4) == 3), 0.0, acc)

    fire(0, 0)

    def pair_body(p, acc):
        t0 = 2 * p
        fire(t0 + 1, 1)
        wait(0)
        acc = compute(t0, 0, acc)

        @pl.when(p + 1 < NG // 2)
        def _():
            fire(t0 + 2, 0)

        wait(1)
        acc = compute(t0 + 1, 1, acc)
        return acc

    lax.fori_loop(0, NG // 2, pair_body, jnp.zeros((L,), jnp.float32))

    pltpu.sync_copy(out_v, out_hbm.at[pl.ds(base, BPW)])


def kernel(user, item, user_table, item_table):
    u2 = user.reshape(NW, BPW)
    i2 = item.reshape(NW, BPW)
    return _mf_sc(u2, i2, user_table.T, item_table.T)


# speedup vs baseline: 3.9678x; 1.0726x over previous
import functools

import jax
import jax.numpy as jnp
from jax import lax
from jax.experimental import pallas as pl
from jax.experimental.pallas import tpu as pltpu
from jax.experimental.pallas import tpu_sc as plsc

L = 16
NC = 2
NS = 16
NW = NC * NS
B = 16384
D = 32
BPW = B // NW
G = 4
NG = BPW // G
WIN = 128

_mesh = plsc.VectorSubcoreMesh(core_axis_name="c", subcore_axis_name="s")


@functools.partial(
    pl.kernel,
    mesh=_mesh,
    out_type=jax.ShapeDtypeStruct((B,), jnp.float32),
    compiler_params=pltpu.CompilerParams(
        needs_layout_passes=False, use_tc_tiling_on_sc=True),
    scratch_types=[
        pltpu.VMEM((BPW + L,), jnp.int32),
        pltpu.VMEM((BPW + L,), jnp.int32),
        pltpu.VMEM((3, G, D, WIN), jnp.float32),
        pltpu.VMEM((3, G, D, WIN), jnp.float32),
        pltpu.VMEM((BPW,), jnp.float32),
        pltpu.SemaphoreType.DMA,
        pltpu.SemaphoreType.DMA,
        pltpu.SemaphoreType.DMA,
    ],
)
def _mf_sc(user_hbm, item_hbm, utab_hbm, itab_hbm, out_hbm,
           uidx_v, iidx_v, ubuf, ibuf, out_v, sem0, sem1, sem2):
    wid = lax.axis_index("s") * NC + lax.axis_index("c")
    base = wid * BPW

    pltpu.sync_copy(user_hbm.at[wid], uidx_v.at[pl.ds(0, BPW)])
    pltpu.sync_copy(item_hbm.at[wid], iidx_v.at[pl.ds(0, BPW)])

    lanes = lax.iota(jnp.int32, L)
    c_lo = lanes
    c_hi = lanes + L
    sems = (sem0, sem1, sem2)

    def fire(t, slab):
        sem = sems[slab]
        gbase = t * G
        ruv = uidx_v[pl.ds(gbase, L)]
        riv = iidx_v[pl.ds(gbase, L)]
        cu = ruv & ~(WIN - 1)
        ci = riv & ~(WIN - 1)
        for k in range(G):
            cuk = pl.multiple_of(cu[k], WIN)
            cik = pl.multiple_of(ci[k], WIN)
            pltpu.async_copy(
                utab_hbm.at[:, pl.ds(cuk, WIN)], ubuf.at[slab, k], sem)
            pltpu.async_copy(
                itab_hbm.at[:, pl.ds(cik, WIN)], ibuf.at[slab, k], sem)

    def wait(slab):
        for _ in range(2 * G):
            pltpu.make_async_copy(
                utab_hbm.at[:, pl.ds(0, WIN)], ubuf.at[0, 0], sems[slab]
            ).wait()

    def compute(t, slab, acc):
        gbase = t * G
        ruv = uidx_v[pl.ds(gbase, L)]
        riv = iidx_v[pl.ds(gbase, L)]
        rsu = ruv & (WIN - 1)
        rsi = riv & (WIN - 1)
        lane0 = (lax.rem(t, 4) * G).astype(jnp.int32)
        for k in range(G):
            su = jnp.full((L,), rsu[k], jnp.int32)
            si = jnp.full((L,), rsi[k], jnp.int32)
            u0 = plsc.load_gather(ubuf.at[slab, k], [c_lo, su])
            u1 = plsc.load_gather(ubuf.at[slab, k], [c_hi, su])
            i0 = plsc.load_gather(ibuf.at[slab, k], [c_lo, si])
            i1 = plsc.load_gather(ibuf.at[slab, k], [c_hi, si])
            p = u0 * i0 + u1 * i1
            dot = lax.reduce_sum(p, (0,))
            acc = jnp.where(lanes == lane0 + k, jnp.full((L,), dot), acc)
        out_v[pl.ds(lax.div(t, 4) * L, L)] = acc
        return jnp.where(jnp.full((L,), lax.rem(t, 4) == 3), 0.0, acc)

    fire(0, 0)
    fire(1, 1)

    def triple_body(p, acc):
        t = 3 * p
        fire(t + 2, 2)
        wait(0)
        acc = compute(t, 0, acc)
        fire(t + 3, 0)
        wait(1)
        acc = compute(t + 1, 1, acc)
        fire(t + 4, 1)
        wait(2)
        acc = compute(t + 2, 2, acc)
        return acc

    acc = lax.fori_loop(0, (NG - 2) // 3, triple_body,
                        jnp.zeros((L,), jnp.float32))
    wait(0)
    acc = compute(NG - 2, 0, acc)
    wait(1)
    compute(NG - 1, 1, acc)

    pltpu.sync_copy(out_v, out_hbm.at[pl.ds(base, BPW)])


def kernel(user, item, user_table, item_table):
    u2 = user.reshape(NW, BPW)
    i2 = item.reshape(NW, BPW)
    return _mf_sc(u2, i2, user_table.T, item_table.T)
